# Initial kernel scaffold; baseline (speedup 1.0000x reference)
import jax
import jax.numpy as jnp
from jax.experimental import pallas as pl


def kernel(inputs, temp, stochastic, embeddings_weight):
    raise NotImplementedError("write your pallas kernel here")


# speedup vs baseline: 1.6264x; 1.6264x over previous
import functools

import jax
import jax.numpy as jnp
from jax import lax
from jax.experimental import pallas as pl
from jax.experimental.pallas import tpu as pltpu
from jax.experimental.pallas import tpu_sc as plsc

NUM_EMBEDDINGS = 1024
EMBEDDING_DIM = 256
COMMITMENT_COST = 0.25
N_ROWS = 16384
BN = 512
GRID = N_ROWS // BN


def _tc_body(temp_ref, x_ref, w_ref, soft_ref, idx_ref, loss_ref, perp_ref,
             counts_ref, acc_ref):
    i = pl.program_id(0)

    x = x_ref[...]
    w = w_ref[...]

    xsq = jnp.sum(x * x, axis=1, keepdims=True)
    xnorm = jnp.sqrt(xsq)
    fn = x / jnp.maximum(xnorm, 1e-12)

    wsq_o = jnp.sum(w * w, axis=1, keepdims=True)
    wnorm = jnp.sqrt(wsq_o)
    wn = w / jnp.maximum(wnorm, 1e-12)

    fsq = jnp.sum(fn * fn, axis=1, keepdims=True)
    wnsq = jnp.sum(wn * wn, axis=1)

    logits = lax.dot_general(fn, wn, (((1,), (1,)), ((), ())),
                             preferred_element_type=jnp.float32)
    d = fsq + wnsq[None, :] - 2.0 * logits

    t = temp_ref[0]
    s = -d / t
    m = jnp.max(s, axis=1, keepdims=True)
    e = jnp.exp(s - m)
    soft_ref[...] = e / jnp.sum(e, axis=1, keepdims=True)

    dmin = jnp.min(d, axis=1, keepdims=True)
    kiota = lax.broadcasted_iota(jnp.int32, (BN, NUM_EMBEDDINGS), 1)
    idx = jnp.min(jnp.where(d == dmin, kiota, NUM_EMBEDDINGS), axis=1)
    idx_ref[0, 0, :] = idx

    onehot = (kiota == idx[:, None]).astype(jnp.float32)
    cnt = jnp.sum(onehot, axis=0)
    wsq_at = jnp.sum(onehot * wsq_o[:, 0][None, :], axis=1)
    wnorm_at = jnp.sum(onehot * wnorm[:, 0][None, :], axis=1)
    logit_at = jnp.sum(onehot * logits, axis=1)
    e_rows = wsq_at - 2.0 * xnorm[:, 0] * wnorm_at * logit_at + xsq[:, 0]
    e_part = jnp.sum(e_rows)

    @pl.when(i == 0)
    def _init():
        counts_ref[...] = jnp.zeros_like(counts_ref)
        acc_ref[0, 0] = 0.0

    counts_ref[0, :] += cnt
    acc_ref[0, 0] += e_part

    @pl.when(i == GRID - 1)
    def _fini():
        avg = counts_ref[0, :] * (1.0 / N_ROWS)
        perp_ref[0, 0] = jnp.exp(-jnp.sum(avg * jnp.log(avg + 1e-10)))
        loss_ref[0, 0] = acc_ref[0, 0] * (COMMITMENT_COST / (N_ROWS * EMBEDDING_DIM))


def _vq_tc(flat_x, weights, temp, interpret=False):
    return pl.pallas_call(
        _tc_body,
        grid=(GRID,),
        in_specs=[
            pl.BlockSpec(memory_space=pltpu.SMEM),
            pl.BlockSpec((BN, EMBEDDING_DIM), lambda i: (i, 0)),
            pl.BlockSpec((NUM_EMBEDDINGS, EMBEDDING_DIM), lambda i: (0, 0)),
        ],
        out_specs=[
            pl.BlockSpec((BN, NUM_EMBEDDINGS), lambda i: (i, 0)),
            pl.BlockSpec((1, 1, BN), lambda i: (i, 0, 0)),
            pl.BlockSpec(memory_space=pltpu.SMEM),
            pl.BlockSpec(memory_space=pltpu.SMEM),
        ],
        out_shape=[
            jax.ShapeDtypeStruct((N_ROWS, NUM_EMBEDDINGS), jnp.float32),
            jax.ShapeDtypeStruct((GRID, 1, BN), jnp.int32),
            jax.ShapeDtypeStruct((1, 1), jnp.float32),
            jax.ShapeDtypeStruct((1, 1), jnp.float32),
        ],
        scratch_shapes=[
            pltpu.VMEM((1, NUM_EMBEDDINGS), jnp.float32),
            pltpu.SMEM((1, 1), jnp.float32),
        ],
        interpret=interpret,
    )(temp, flat_x, weights)


_NUM_SC = 2
_NUM_SUBCORES = 16
_NW = _NUM_SC * _NUM_SUBCORES
_B_PER_W = N_ROWS // _NW
_CHUNK = 128
_NCHUNK = _B_PER_W // _CHUNK


def _sc_gather_body(table_hbm, idx_hbm, out_hbm, idx0, idx1, rows0, rows1,
                    sem0, sem1):
    wid = lax.axis_index("s") * _NUM_SC + lax.axis_index("c")
    base = wid * _B_PER_W
    idxb = (idx0, idx1)
    rowsb = (rows0, rows1)
    semb = (sem0, sem1)
    cps = [None, None]
    for c in range(_NCHUNK):
        b = c & 1
        if cps[b] is not None:
            cps[b].wait()
            pltpu.sync_copy(rowsb[b],
                            out_hbm.at[pl.ds(base + (c - 2) * _CHUNK, _CHUNK)])
        pltpu.sync_copy(idx_hbm.at[pl.ds(base + c * _CHUNK, _CHUNK)], idxb[b])
        cps[b] = pltpu.async_copy(table_hbm.at[idxb[b]], rowsb[b], semb[b])
    for c in range(_NCHUNK - 2, _NCHUNK):
        b = c & 1
        cps[b].wait()
        pltpu.sync_copy(rowsb[b],
                        out_hbm.at[pl.ds(base + c * _CHUNK, _CHUNK)])


@functools.lru_cache(maxsize=1)
def _sc_gather_kernel():
    return pl.kernel(
        _sc_gather_body,
        mesh=plsc.VectorSubcoreMesh(core_axis_name="c", subcore_axis_name="s",
                                    num_cores=_NUM_SC,
                                    num_subcores=_NUM_SUBCORES),
        out_type=jax.ShapeDtypeStruct((N_ROWS, EMBEDDING_DIM), jnp.float32),
        scratch_types=[
            pltpu.VMEM((_CHUNK,), jnp.int32),
            pltpu.VMEM((_CHUNK,), jnp.int32),
            pltpu.VMEM((_CHUNK, EMBEDDING_DIM), jnp.float32),
            pltpu.VMEM((_CHUNK, EMBEDDING_DIM), jnp.float32),
            pltpu.SemaphoreType.DMA,
            pltpu.SemaphoreType.DMA,
        ],
    )


def kernel(inputs, temp, stochastic, embeddings_weight):
    bs, channel = inputs.shape[0], inputs.shape[1]
    x = jnp.transpose(inputs, (0, 2, 3, 1))
    input_shape = x.shape
    flat_x = x.reshape(-1, EMBEDDING_DIM)

    temp_arr = jnp.asarray(temp, jnp.float32).reshape(1)

    soft, idx3, loss, perp = _vq_tc(flat_x, embeddings_weight, temp_arr)
    idx_flat = idx3.reshape(N_ROWS)

    q_flat = _sc_gather_kernel()(embeddings_weight, idx_flat)

    quantized = q_flat.reshape(input_shape)
    quantized = jnp.transpose(quantized, (0, 3, 2, 1))

    encoding_indices = idx_flat.reshape(N_ROWS, 1)
    soft_codes = soft.reshape(bs, channel, -1)
    return (quantized, loss[0, 0], perp[0, 0], encoding_indices, soft_codes)


# speedup vs baseline: 1.7137x; 1.0537x over previous
import functools

import jax
import jax.numpy as jnp
from jax import lax
from jax.experimental import pallas as pl
from jax.experimental.pallas import tpu as pltpu
from jax.experimental.pallas import tpu_sc as plsc

NUM_EMBEDDINGS = 1024
EMBEDDING_DIM = 256
COMMITMENT_COST = 0.25
N_ROWS = 16384
BN = 512
GRID = N_ROWS // BN


def _tc_body(temp_ref, x_ref, w_ref, soft_ref, idx_ref, loss_ref, perp_ref,
             counts_ref, acc_ref, wn_ref, rtwnsq_ref, vg_ref):
    i = pl.program_id(0)
    t = temp_ref[0]
    rt = 1.0 / t

    @pl.when(i == 0)
    def _init():
        w = w_ref[...]
        wsq_o = jnp.sum(w * w, axis=1, keepdims=True)
        wnorm = jnp.sqrt(wsq_o)
        wn = w / jnp.maximum(wnorm, 1e-12)
        wnsq = jnp.sum(wn * wn, axis=1, keepdims=True)
        wn_ref[...] = wn
        rtwnsq_ref[0, :] = rt * wnsq[:, 0]
        vg_ref[...] = jnp.concatenate(
            [wsq_o, wnorm, wnsq,
             jnp.zeros((NUM_EMBEDDINGS, 5), jnp.float32)], axis=1)
        counts_ref[...] = jnp.zeros_like(counts_ref)
        acc_ref[0, 0] = 0.0

    x = x_ref[...]
    xsq = jnp.sum(x * x, axis=1, keepdims=True)
    xnorm = jnp.sqrt(xsq)
    fn2 = x * ((2.0 * rt) / jnp.maximum(xnorm, 1e-12))

    raw = lax.dot_general(fn2, wn_ref[...], (((1,), (1,)), ((), ())),
                          preferred_element_type=jnp.float32)
    u = raw - rtwnsq_ref[0, :][None, :]

    e = jnp.exp(u)
    denom = jnp.sum(e, axis=1, keepdims=True)
    soft_ref[...] = e * (1.0 / denom)

    m = jnp.max(u, axis=1, keepdims=True)
    kiota = lax.broadcasted_iota(jnp.int32, (BN, NUM_EMBEDDINGS), 1)
    idx = jnp.min(jnp.where(u == m, kiota, NUM_EMBEDDINGS), axis=1)
    idx_ref[0, 0, :] = idx

    onehot = (kiota == idx[:, None]).astype(jnp.float32)
    cnt = lax.dot_general(jnp.ones((1, BN), jnp.float32), onehot,
                          (((1,), (0,)), ((), ())),
                          preferred_element_type=jnp.float32)
    g = lax.dot_general(onehot, vg_ref[...], (((1,), (0,)), ((), ())),
                        preferred_element_type=jnp.float32)
    wsq_at = g[:, 0:1]
    wnorm_at = g[:, 1:2]
    wnsq_at = g[:, 2:3]
    e_rows = wsq_at - xnorm * wnorm_at * (t * m + wnsq_at) + xsq
    counts_ref[...] += cnt
    acc_ref[0, 0] += jnp.sum(e_rows)

    @pl.when(i == GRID - 1)
    def _fini():
        avg = counts_ref[0, :] * (1.0 / N_ROWS)
        perp_ref[0, 0] = jnp.exp(-jnp.sum(avg * jnp.log(avg + 1e-10)))
        loss_ref[0, 0] = acc_ref[0, 0] * (COMMITMENT_COST / (N_ROWS * EMBEDDING_DIM))


def _vq_tc(flat_x, weights, temp, interpret=False):
    return pl.pallas_call(
        _tc_body,
        grid=(GRID,),
        in_specs=[
            pl.BlockSpec(memory_space=pltpu.SMEM),
            pl.BlockSpec((BN, EMBEDDING_DIM), lambda i: (i, 0)),
            pl.BlockSpec((NUM_EMBEDDINGS, EMBEDDING_DIM), lambda i: (0, 0)),
        ],
        out_specs=[
            pl.BlockSpec((BN, NUM_EMBEDDINGS), lambda i: (i, 0)),
            pl.BlockSpec((1, 1, BN), lambda i: (i, 0, 0)),
            pl.BlockSpec(memory_space=pltpu.SMEM),
            pl.BlockSpec(memory_space=pltpu.SMEM),
        ],
        out_shape=[
            jax.ShapeDtypeStruct((N_ROWS, NUM_EMBEDDINGS), jnp.float32),
            jax.ShapeDtypeStruct((GRID, 1, BN), jnp.int32),
            jax.ShapeDtypeStruct((1, 1), jnp.float32),
            jax.ShapeDtypeStruct((1, 1), jnp.float32),
        ],
        scratch_shapes=[
            pltpu.VMEM((1, NUM_EMBEDDINGS), jnp.float32),
            pltpu.SMEM((1, 1), jnp.float32),
            pltpu.VMEM((NUM_EMBEDDINGS, EMBEDDING_DIM), jnp.float32),
            pltpu.VMEM((1, NUM_EMBEDDINGS), jnp.float32),
            pltpu.VMEM((NUM_EMBEDDINGS, 8), jnp.float32),
        ],
        interpret=interpret,
    )(temp, flat_x, weights)


_NUM_SC = 2
_NUM_SUBCORES = 16
_NW = _NUM_SC * _NUM_SUBCORES
_B_PER_W = N_ROWS // _NW
_CHUNK = 128
_NCHUNK = _B_PER_W // _CHUNK


def _sc_gather_body(table_hbm, idx_hbm, out_hbm, idx0, idx1, rows0, rows1,
                    sem0, sem1):
    wid = lax.axis_index("s") * _NUM_SC + lax.axis_index("c")
    base = wid * _B_PER_W
    idxb = (idx0, idx1)
    rowsb = (rows0, rows1)
    semb = (sem0, sem1)
    cps = [None, None]
    for c in range(_NCHUNK):
        b = c & 1
        if cps[b] is not None:
            cps[b].wait()
            pltpu.sync_copy(rowsb[b],
                            out_hbm.at[pl.ds(base + (c - 2) * _CHUNK, _CHUNK)])
        pltpu.sync_copy(idx_hbm.at[pl.ds(base + c * _CHUNK, _CHUNK)], idxb[b])
        cps[b] = pltpu.async_copy(table_hbm.at[idxb[b]], rowsb[b], semb[b])
    for c in range(_NCHUNK - 2, _NCHUNK):
        b = c & 1
        cps[b].wait()
        pltpu.sync_copy(rowsb[b],
                        out_hbm.at[pl.ds(base + c * _CHUNK, _CHUNK)])


@functools.lru_cache(maxsize=1)
def _sc_gather_kernel():
    return pl.kernel(
        _sc_gather_body,
        mesh=plsc.VectorSubcoreMesh(core_axis_name="c", subcore_axis_name="s",
                                    num_cores=_NUM_SC,
                                    num_subcores=_NUM_SUBCORES),
        out_type=jax.ShapeDtypeStruct((N_ROWS, EMBEDDING_DIM), jnp.float32),
        scratch_types=[
            pltpu.VMEM((_CHUNK,), jnp.int32),
            pltpu.VMEM((_CHUNK,), jnp.int32),
            pltpu.VMEM((_CHUNK, EMBEDDING_DIM), jnp.float32),
            pltpu.VMEM((_CHUNK, EMBEDDING_DIM), jnp.float32),
            pltpu.SemaphoreType.DMA,
            pltpu.SemaphoreType.DMA,
        ],
    )


def kernel(inputs, temp, stochastic, embeddings_weight):
    bs, channel = inputs.shape[0], inputs.shape[1]
    x = jnp.transpose(inputs, (0, 2, 3, 1))
    input_shape = x.shape
    flat_x = x.reshape(-1, EMBEDDING_DIM)

    temp_arr = jnp.asarray(temp, jnp.float32).reshape(1)

    soft, idx3, loss, perp = _vq_tc(flat_x, embeddings_weight, temp_arr)
    idx_flat = idx3.reshape(N_ROWS)

    q_flat = _sc_gather_kernel()(embeddings_weight, idx_flat)

    quantized = q_flat.reshape(input_shape)
    quantized = jnp.transpose(quantized, (0, 3, 2, 1))

    encoding_indices = idx_flat.reshape(N_ROWS, 1)
    soft_codes = soft.reshape(bs, channel, -1)
    return (quantized, loss[0, 0], perp[0, 0], encoding_indices, soft_codes)


# speedup vs baseline: 2.3924x; 1.3960x over previous
import functools

import jax
import jax.numpy as jnp
from jax import lax
from jax.experimental import pallas as pl
from jax.experimental.pallas import tpu as pltpu
from jax.experimental.pallas import tpu_sc as plsc

NUM_EMBEDDINGS = 1024
EMBEDDING_DIM = 256
COMMITMENT_COST = 0.25
N_ROWS = 16384
BN = 1024
GRID = N_ROWS // BN


def _tc_body(temp_ref, x_ref, w_ref, soft_ref, idx_ref, loss_ref, perp_ref,
             counts_ref, acc_ref, wn_ref, rtwnsq_ref, vg_ref):
    i = pl.program_id(0)
    t = temp_ref[0]
    rt = 1.0 / t

    @pl.when(i == 0)
    def _init():
        w = w_ref[...]
        wsq_o = jnp.sum(w * w, axis=1, keepdims=True)
        wnorm = jnp.sqrt(wsq_o)
        wn = w / jnp.maximum(wnorm, 1e-12)
        wnsq = jnp.sum(wn * wn, axis=1, keepdims=True)
        wn_ref[...] = wn
        rtwnsq_ref[0, :] = rt * wnsq[:, 0]
        vg_ref[...] = jnp.concatenate(
            [wsq_o, wnorm, wnsq,
             jnp.zeros((NUM_EMBEDDINGS, 5), jnp.float32)], axis=1)
        counts_ref[...] = jnp.zeros_like(counts_ref)
        acc_ref[0, 0] = 0.0

    x = x_ref[...]
    xsq = jnp.sum(x * x, axis=1, keepdims=True)
    xnorm = jnp.sqrt(xsq)
    fn2 = x * ((2.0 * rt) / jnp.maximum(xnorm, 1e-12))

    raw = lax.dot_general(fn2, wn_ref[...], (((1,), (1,)), ((), ())),
                          preferred_element_type=jnp.float32)
    u = raw - rtwnsq_ref[0, :][None, :]

    e = jnp.exp(u)
    denom = jnp.sum(e, axis=1, keepdims=True)
    en = e * (1.0 / denom)
    for r in range(4):
        soft_ref[0, :, r * NUM_EMBEDDINGS:(r + 1) * NUM_EMBEDDINGS] = (
            en[r * 256:(r + 1) * 256, :])

    m = jnp.max(u, axis=1, keepdims=True)
    mask = u == m
    kiota = lax.broadcasted_iota(jnp.int32, (BN, NUM_EMBEDDINGS), 1)
    idx = jnp.min(jnp.where(mask, kiota, NUM_EMBEDDINGS), axis=1)
    idx_ref[0, 0, :] = idx

    maskf = mask.astype(jnp.float32)
    cnt = lax.dot_general(jnp.ones((1, BN), jnp.float32), maskf,
                          (((1,), (0,)), ((), ())),
                          preferred_element_type=jnp.float32)
    g = lax.dot_general(maskf, vg_ref[...], (((1,), (0,)), ((), ())),
                        preferred_element_type=jnp.float32)
    wsq_at = g[:, 0:1]
    wnorm_at = g[:, 1:2]
    wnsq_at = g[:, 2:3]
    e_rows = wsq_at - xnorm * wnorm_at * (t * m + wnsq_at) + xsq
    counts_ref[...] += cnt
    acc_ref[0, 0] += jnp.sum(e_rows)

    @pl.when(i == GRID - 1)
    def _fini():
        avg = counts_ref[0, :] * (1.0 / N_ROWS)
        perp_ref[0, 0] = jnp.exp(-jnp.sum(avg * jnp.log(avg + 1e-10)))
        loss_ref[0, 0] = acc_ref[0, 0] * (COMMITMENT_COST / (N_ROWS * EMBEDDING_DIM))


def _vq_tc(xp, weights, temp, interpret=False):
    return pl.pallas_call(
        _tc_body,
        grid=(GRID,),
        in_specs=[
            pl.BlockSpec(memory_space=pltpu.SMEM),
            pl.BlockSpec((BN, EMBEDDING_DIM), lambda i: (i, 0)),
            pl.BlockSpec((NUM_EMBEDDINGS, EMBEDDING_DIM), lambda i: (0, 0)),
        ],
        out_specs=[
            pl.BlockSpec((1, 256, 4 * NUM_EMBEDDINGS), lambda i: (i, 0, 0)),
            pl.BlockSpec((1, 1, BN), lambda i: (i, 0, 0)),
            pl.BlockSpec(memory_space=pltpu.SMEM),
            pl.BlockSpec(memory_space=pltpu.SMEM),
        ],
        out_shape=[
            jax.ShapeDtypeStruct((GRID, 256, 4 * NUM_EMBEDDINGS), jnp.float32),
            jax.ShapeDtypeStruct((GRID, 1, BN), jnp.int32),
            jax.ShapeDtypeStruct((1, 1), jnp.float32),
            jax.ShapeDtypeStruct((1, 1), jnp.float32),
        ],
        scratch_shapes=[
            pltpu.VMEM((1, NUM_EMBEDDINGS), jnp.float32),
            pltpu.SMEM((1, 1), jnp.float32),
            pltpu.VMEM((NUM_EMBEDDINGS, EMBEDDING_DIM), jnp.float32),
            pltpu.VMEM((1, NUM_EMBEDDINGS), jnp.float32),
            pltpu.VMEM((NUM_EMBEDDINGS, 8), jnp.float32),
        ],
        interpret=interpret,
    )(temp, xp, weights)


_NUM_SC = 2
_NUM_SUBCORES = 16
_NW = _NUM_SC * _NUM_SUBCORES
_B_PER_W = N_ROWS // _NW
_CHUNK = 128
_NCHUNK = _B_PER_W // _CHUNK


def _sc_gather_body(table_hbm, idx_hbm, out_hbm, idx0, idx1, rows0, rows1,
                    sem0, sem1):
    wid = lax.axis_index("s") * _NUM_SC + lax.axis_index("c")
    base = wid * _B_PER_W
    idxb = (idx0, idx1)
    rowsb = (rows0, rows1)
    semb = (sem0, sem1)
    cps = [None, None]
    for c in range(_NCHUNK):
        b = c & 1
        if cps[b] is not None:
            cps[b].wait()
            pltpu.sync_copy(rowsb[b],
                            out_hbm.at[pl.ds(base + (c - 2) * _CHUNK, _CHUNK)])
        pltpu.sync_copy(idx_hbm.at[pl.ds(base + c * _CHUNK, _CHUNK)], idxb[b])
        cps[b] = pltpu.async_copy(table_hbm.at[idxb[b]], rowsb[b], semb[b])
    for c in range(_NCHUNK - 2, _NCHUNK):
        b = c & 1
        cps[b].wait()
        pltpu.sync_copy(rowsb[b],
                        out_hbm.at[pl.ds(base + c * _CHUNK, _CHUNK)])


@functools.lru_cache(maxsize=1)
def _sc_gather_kernel():
    return pl.kernel(
        _sc_gather_body,
        mesh=plsc.VectorSubcoreMesh(core_axis_name="c", subcore_axis_name="s",
                                    num_cores=_NUM_SC,
                                    num_subcores=_NUM_SUBCORES),
        out_type=jax.ShapeDtypeStruct((N_ROWS, EMBEDDING_DIM), jnp.float32),
        scratch_types=[
            pltpu.VMEM((_CHUNK,), jnp.int32),
            pltpu.VMEM((_CHUNK,), jnp.int32),
            pltpu.VMEM((_CHUNK, EMBEDDING_DIM), jnp.float32),
            pltpu.VMEM((_CHUNK, EMBEDDING_DIM), jnp.float32),
            pltpu.SemaphoreType.DMA,
            pltpu.SemaphoreType.DMA,
        ],
    )


def kernel(inputs, temp, stochastic, embeddings_weight):
    bs, channel = inputs.shape[0], inputs.shape[1]
    xp = (inputs.reshape(bs, channel, 32, 8, 4)
          .transpose(0, 4, 2, 3, 1)
          .reshape(N_ROWS, EMBEDDING_DIM))

    temp_arr = jnp.asarray(temp, jnp.float32).reshape(1)

    soft_codes, idx3, loss, perp = _vq_tc(xp, embeddings_weight, temp_arr)

    idxp = idx3.reshape(bs, 4, 32, 8)
    idx_flat = idxp.transpose(0, 2, 3, 1).reshape(N_ROWS)
    idx_wh = idxp.transpose(0, 3, 1, 2).reshape(N_ROWS)

    q = _sc_gather_kernel()(embeddings_weight, idx_wh)
    quantized = jnp.transpose(q.reshape(bs, 32, 32, EMBEDDING_DIM),
                              (0, 3, 1, 2))

    encoding_indices = idx_flat.reshape(N_ROWS, 1)
    return (quantized, loss[0, 0], perp[0, 0], encoding_indices, soft_codes)


# speedup vs baseline: 2.4260x; 1.0140x over previous
import functools

import jax
import jax.numpy as jnp
from jax import lax
from jax.experimental import pallas as pl
from jax.experimental.pallas import tpu as pltpu
from jax.experimental.pallas import tpu_sc as plsc

NUM_EMBEDDINGS = 1024
EMBEDDING_DIM = 256
COMMITMENT_COST = 0.25
N_ROWS = 16384
BN = 1024
GRID = N_ROWS // BN


def _tc_body(temp_ref, x_ref, w_ref, soft_ref, idx_ref, loss_ref, perp_ref,
             counts_ref, acc_ref, wn_ref, rtwnsq_ref, vg_ref):
    i = pl.program_id(0)
    t = temp_ref[0]
    rt = 1.0 / t

    @pl.when(i == 0)
    def _init():
        w = w_ref[...]
        wsq_o = jnp.sum(w * w, axis=1, keepdims=True)
        wnorm = jnp.sqrt(wsq_o)
        wn = w / jnp.maximum(wnorm, 1e-12)
        wnsq = jnp.sum(wn * wn, axis=1, keepdims=True)
        wn_ref[...] = wn
        rtwnsq_ref[0, :] = rt * wnsq[:, 0]
        kcol = lax.broadcasted_iota(jnp.int32, (NUM_EMBEDDINGS, 1), 0).astype(
            jnp.float32)
        vg_ref[...] = jnp.concatenate(
            [wsq_o, wnorm, wnsq, kcol,
             jnp.ones((NUM_EMBEDDINGS, 1), jnp.float32),
             jnp.zeros((NUM_EMBEDDINGS, 3), jnp.float32)], axis=1)
        counts_ref[...] = jnp.zeros_like(counts_ref)
        acc_ref[0, 0] = 0.0

    x = x_ref[...]
    xsq = jnp.sum(x * x, axis=1, keepdims=True)
    xnorm = jnp.sqrt(xsq)
    fn2 = x * ((2.0 * rt) / jnp.maximum(xnorm, 1e-12))

    raw = lax.dot_general(fn2, wn_ref[...], (((1,), (1,)), ((), ())),
                          preferred_element_type=jnp.float32)
    u = raw - rtwnsq_ref[0, :][None, :]

    e = jnp.exp(u)
    denom = lax.dot_general(e, jnp.ones((NUM_EMBEDDINGS, 1), jnp.float32),
                            (((1,), (0,)), ((), ())),
                            preferred_element_type=jnp.float32)
    en = e * (1.0 / denom)
    for r in range(4):
        soft_ref[0, :, r * NUM_EMBEDDINGS:(r + 1) * NUM_EMBEDDINGS] = (
            en[r * 256:(r + 1) * 256, :])

    m = jnp.max(u, axis=1, keepdims=True)
    mask = u == m
    maskf = mask.astype(jnp.float32)
    cnt = lax.dot_general(jnp.ones((1, BN), jnp.float32), maskf,
                          (((1,), (0,)), ((), ())),
                          preferred_element_type=jnp.float32)
    g = lax.dot_general(maskf, vg_ref[...], (((1,), (0,)), ((), ())),
                        preferred_element_type=jnp.float32)
    wsq_at = g[:, 0:1]
    wnorm_at = g[:, 1:2]
    wnsq_at = g[:, 2:3]
    idx_ref[0, 0, :] = g[:, 3].astype(jnp.int32)

    @pl.when(jnp.max(g[:, 4]) > 1.5)
    def _tie_fallback():
        kiota = lax.broadcasted_iota(jnp.int32, (BN, NUM_EMBEDDINGS), 1)
        idx_ref[0, 0, :] = jnp.min(
            jnp.where(mask, kiota, NUM_EMBEDDINGS), axis=1)
    e_rows = wsq_at - xnorm * wnorm_at * (t * m + wnsq_at) + xsq
    counts_ref[...] += cnt
    acc_ref[0, 0] += jnp.sum(e_rows)

    @pl.when(i == GRID - 1)
    def _fini():
        avg = counts_ref[0, :] * (1.0 / N_ROWS)
        perp_ref[0, 0] = jnp.exp(-jnp.sum(avg * jnp.log(avg + 1e-10)))
        loss_ref[0, 0] = acc_ref[0, 0] * (COMMITMENT_COST / (N_ROWS * EMBEDDING_DIM))


def _vq_tc(xp, weights, temp, interpret=False):
    return pl.pallas_call(
        _tc_body,
        grid=(GRID,),
        in_specs=[
            pl.BlockSpec(memory_space=pltpu.SMEM),
            pl.BlockSpec((BN, EMBEDDING_DIM), lambda i: (i, 0)),
            pl.BlockSpec((NUM_EMBEDDINGS, EMBEDDING_DIM), lambda i: (0, 0)),
        ],
        out_specs=[
            pl.BlockSpec((1, 256, 4 * NUM_EMBEDDINGS), lambda i: (i, 0, 0)),
            pl.BlockSpec((1, 1, BN), lambda i: (i, 0, 0)),
            pl.BlockSpec(memory_space=pltpu.SMEM),
            pl.BlockSpec(memory_space=pltpu.SMEM),
        ],
        out_shape=[
            jax.ShapeDtypeStruct((GRID, 256, 4 * NUM_EMBEDDINGS), jnp.float32),
            jax.ShapeDtypeStruct((GRID, 1, BN), jnp.int32),
            jax.ShapeDtypeStruct((1, 1), jnp.float32),
            jax.ShapeDtypeStruct((1, 1), jnp.float32),
        ],
        scratch_shapes=[
            pltpu.VMEM((1, NUM_EMBEDDINGS), jnp.float32),
            pltpu.SMEM((1, 1), jnp.float32),
            pltpu.VMEM((NUM_EMBEDDINGS, EMBEDDING_DIM), jnp.float32),
            pltpu.VMEM((1, NUM_EMBEDDINGS), jnp.float32),
            pltpu.VMEM((NUM_EMBEDDINGS, 8), jnp.float32),
        ],
        interpret=interpret,
    )(temp, xp, weights)


_NUM_SC = 2
_NUM_SUBCORES = 16
_NW = _NUM_SC * _NUM_SUBCORES
_B_PER_W = N_ROWS // _NW
_CHUNK = 128
_NCHUNK = _B_PER_W // _CHUNK


def _sc_gather_body(table_hbm, idx_hbm, out_hbm, idx0, idx1, rows0, rows1,
                    sem0, sem1):
    wid = lax.axis_index("s") * _NUM_SC + lax.axis_index("c")
    base = wid * _B_PER_W
    idxb = (idx0, idx1)
    rowsb = (rows0, rows1)
    semb = (sem0, sem1)
    cps = [None, None]
    for c in range(_NCHUNK):
        b = c & 1
        if cps[b] is not None:
            cps[b].wait()
            pltpu.sync_copy(rowsb[b],
                            out_hbm.at[pl.ds(base + (c - 2) * _CHUNK, _CHUNK)])
        pltpu.sync_copy(idx_hbm.at[pl.ds(base + c * _CHUNK, _CHUNK)], idxb[b])
        cps[b] = pltpu.async_copy(table_hbm.at[idxb[b]], rowsb[b], semb[b])
    for c in range(_NCHUNK - 2, _NCHUNK):
        b = c & 1
        cps[b].wait()
        pltpu.sync_copy(rowsb[b],
                        out_hbm.at[pl.ds(base + c * _CHUNK, _CHUNK)])


@functools.lru_cache(maxsize=1)
def _sc_gather_kernel():
    return pl.kernel(
        _sc_gather_body,
        mesh=plsc.VectorSubcoreMesh(core_axis_name="c", subcore_axis_name="s",
                                    num_cores=_NUM_SC,
                                    num_subcores=_NUM_SUBCORES),
        out_type=jax.ShapeDtypeStruct((N_ROWS, EMBEDDING_DIM), jnp.float32),
        scratch_types=[
            pltpu.VMEM((_CHUNK,), jnp.int32),
            pltpu.VMEM((_CHUNK,), jnp.int32),
            pltpu.VMEM((_CHUNK, EMBEDDING_DIM), jnp.float32),
            pltpu.VMEM((_CHUNK, EMBEDDING_DIM), jnp.float32),
            pltpu.SemaphoreType.DMA,
            pltpu.SemaphoreType.DMA,
        ],
    )


def kernel(inputs, temp, stochastic, embeddings_weight):
    bs, channel = inputs.shape[0], inputs.shape[1]
    xp = (inputs.reshape(bs, channel, 32, 8, 4)
          .transpose(0, 4, 2, 3, 1)
          .reshape(N_ROWS, EMBEDDING_DIM))

    temp_arr = jnp.asarray(temp, jnp.float32).reshape(1)

    soft_codes, idx3, loss, perp = _vq_tc(xp, embeddings_weight, temp_arr)

    idxp = idx3.reshape(bs, 4, 32, 8)
    idx_flat = idxp.transpose(0, 2, 3, 1).reshape(N_ROWS)
    idx_wh = idxp.transpose(0, 3, 1, 2).reshape(N_ROWS)

    q = _sc_gather_kernel()(embeddings_weight, idx_wh)
    quantized = jnp.transpose(q.reshape(bs, 32, 32, EMBEDDING_DIM),
                              (0, 3, 1, 2))

    encoding_indices = idx_flat.reshape(N_ROWS, 1)
    return (quantized, loss[0, 0], perp[0, 0], encoding_indices, soft_codes)


# speedup vs baseline: 2.5007x; 1.0308x over previous
import functools

import jax
import jax.numpy as jnp
from jax import lax
from jax.experimental import pallas as pl
from jax.experimental.pallas import tpu as pltpu
from jax.experimental.pallas import tpu_sc as plsc

NUM_EMBEDDINGS = 1024
EMBEDDING_DIM = 256
COMMITMENT_COST = 0.25
N_ROWS = 16384
BN = 1024
GRID = N_ROWS // BN


def _tc_body(temp_ref, x_ref, w_ref, soft_ref, idx_ref, loss_ref, perp_ref,
             counts_ref, acc_ref, wn_ref, rtwnsq_ref, vg_ref):
    i = pl.program_id(0)
    t = temp_ref[0]
    rt = 1.0 / t

    @pl.when(i == 0)
    def _init():
        w = w_ref[...]
        wsq_o = jnp.sum(w * w, axis=1, keepdims=True)
        wnorm = jnp.sqrt(wsq_o)
        wn = w / jnp.maximum(wnorm, 1e-12)
        wnsq = jnp.sum(wn * wn, axis=1, keepdims=True)
        wn_ref[...] = wn
        rtwnsq_ref[0, :] = rt * wnsq[:, 0]
        kcol = lax.broadcasted_iota(jnp.int32, (NUM_EMBEDDINGS, 1), 0).astype(
            jnp.float32)
        vg_ref[...] = jnp.concatenate(
            [wsq_o, wnorm, wnsq, kcol,
             jnp.ones((NUM_EMBEDDINGS, 1), jnp.float32),
             jnp.zeros((NUM_EMBEDDINGS, 3), jnp.float32)], axis=1)
        counts_ref[...] = jnp.zeros_like(counts_ref)
        acc_ref[0, 0] = 0.0

    x = x_ref[...]
    xsq = jnp.sum(x * x, axis=1, keepdims=True)
    xnorm = jnp.sqrt(xsq)
    fn2 = x * ((2.0 * rt) / jnp.maximum(xnorm, 1e-12))

    raw = lax.dot_general(fn2, wn_ref[...], (((1,), (1,)), ((), ())),
                          preferred_element_type=jnp.float32)
    u = raw - rtwnsq_ref[0, :][None, :]

    e = jnp.exp(u)
    denom = lax.dot_general(e, jnp.ones((NUM_EMBEDDINGS, 1), jnp.float32),
                            (((1,), (0,)), ((), ())),
                            preferred_element_type=jnp.float32)
    en = e * (1.0 / denom)
    for r in range(4):
        soft_ref[0, :, r * NUM_EMBEDDINGS:(r + 1) * NUM_EMBEDDINGS] = (
            en[r * 256:(r + 1) * 256, :])

    m = jnp.max(u, axis=1, keepdims=True)
    mask = u == m
    maskf = mask.astype(jnp.float32)
    cnt = lax.dot_general(jnp.ones((1, BN), jnp.float32), maskf,
                          (((1,), (0,)), ((), ())),
                          preferred_element_type=jnp.float32)
    g = lax.dot_general(maskf, vg_ref[...], (((1,), (0,)), ((), ())),
                        preferred_element_type=jnp.float32)
    wsq_at = g[:, 0:1]
    wnorm_at = g[:, 1:2]
    wnsq_at = g[:, 2:3]
    idx_ref[0, 0, :] = jnp.argmax(u, axis=1).astype(jnp.int32)
    e_rows = wsq_at - xnorm * wnorm_at * (t * m + wnsq_at) + xsq
    counts_ref[...] += cnt
    acc_ref[0, 0] += jnp.sum(e_rows)

    @pl.when(i == GRID - 1)
    def _fini():
        avg = counts_ref[0, :] * (1.0 / N_ROWS)
        perp_ref[0, 0] = jnp.exp(-jnp.sum(avg * jnp.log(avg + 1e-10)))
        loss_ref[0, 0] = acc_ref[0, 0] * (COMMITMENT_COST / (N_ROWS * EMBEDDING_DIM))


def _vq_tc(xp, weights, temp, interpret=False):
    return pl.pallas_call(
        _tc_body,
        grid=(GRID,),
        in_specs=[
            pl.BlockSpec(memory_space=pltpu.SMEM),
            pl.BlockSpec((BN, EMBEDDING_DIM), lambda i: (i, 0)),
            pl.BlockSpec((NUM_EMBEDDINGS, EMBEDDING_DIM), lambda i: (0, 0)),
        ],
        out_specs=[
            pl.BlockSpec((1, 256, 4 * NUM_EMBEDDINGS), lambda i: (i, 0, 0)),
            pl.BlockSpec((1, 1, BN), lambda i: (i, 0, 0)),
            pl.BlockSpec(memory_space=pltpu.SMEM),
            pl.BlockSpec(memory_space=pltpu.SMEM),
        ],
        out_shape=[
            jax.ShapeDtypeStruct((GRID, 256, 4 * NUM_EMBEDDINGS), jnp.float32),
            jax.ShapeDtypeStruct((GRID, 1, BN), jnp.int32),
            jax.ShapeDtypeStruct((1, 1), jnp.float32),
            jax.ShapeDtypeStruct((1, 1), jnp.float32),
        ],
        scratch_shapes=[
            pltpu.VMEM((1, NUM_EMBEDDINGS), jnp.float32),
            pltpu.SMEM((1, 1), jnp.float32),
            pltpu.VMEM((NUM_EMBEDDINGS, EMBEDDING_DIM), jnp.float32),
            pltpu.VMEM((1, NUM_EMBEDDINGS), jnp.float32),
            pltpu.VMEM((NUM_EMBEDDINGS, 8), jnp.float32),
        ],
        interpret=interpret,
    )(temp, xp, weights)


_NUM_SC = 2
_NUM_SUBCORES = 16
_NW = _NUM_SC * _NUM_SUBCORES
_B_PER_W = N_ROWS // _NW
_CHUNK = 128
_NCHUNK = _B_PER_W // _CHUNK


def _sc_gather_body(table_hbm, idx_hbm, out_hbm, idx0, idx1, rows0, rows1,
                    sem0, sem1):
    wid = lax.axis_index("s") * _NUM_SC + lax.axis_index("c")
    base = wid * _B_PER_W
    idxb = (idx0, idx1)
    rowsb = (rows0, rows1)
    semb = (sem0, sem1)
    cps = [None, None]
    for c in range(_NCHUNK):
        b = c & 1
        if cps[b] is not None:
            cps[b].wait()
            pltpu.sync_copy(rowsb[b],
                            out_hbm.at[pl.ds(base + (c - 2) * _CHUNK, _CHUNK)])
        pltpu.sync_copy(idx_hbm.at[pl.ds(base + c * _CHUNK, _CHUNK)], idxb[b])
        cps[b] = pltpu.async_copy(table_hbm.at[idxb[b]], rowsb[b], semb[b])
    for c in range(_NCHUNK - 2, _NCHUNK):
        b = c & 1
        cps[b].wait()
        pltpu.sync_copy(rowsb[b],
                        out_hbm.at[pl.ds(base + c * _CHUNK, _CHUNK)])


@functools.lru_cache(maxsize=1)
def _sc_gather_kernel():
    return pl.kernel(
        _sc_gather_body,
        mesh=plsc.VectorSubcoreMesh(core_axis_name="c", subcore_axis_name="s",
                                    num_cores=_NUM_SC,
                                    num_subcores=_NUM_SUBCORES),
        out_type=jax.ShapeDtypeStruct((N_ROWS, EMBEDDING_DIM), jnp.float32),
        scratch_types=[
            pltpu.VMEM((_CHUNK,), jnp.int32),
            pltpu.VMEM((_CHUNK,), jnp.int32),
            pltpu.VMEM((_CHUNK, EMBEDDING_DIM), jnp.float32),
            pltpu.VMEM((_CHUNK, EMBEDDING_DIM), jnp.float32),
            pltpu.SemaphoreType.DMA,
            pltpu.SemaphoreType.DMA,
        ],
    )


def kernel(inputs, temp, stochastic, embeddings_weight):
    bs, channel = inputs.shape[0], inputs.shape[1]
    xp = (inputs.reshape(bs, channel, 32, 8, 4)
          .transpose(0, 4, 2, 3, 1)
          .reshape(N_ROWS, EMBEDDING_DIM))

    temp_arr = jnp.asarray(temp, jnp.float32).reshape(1)

    soft_codes, idx3, loss, perp = _vq_tc(xp, embeddings_weight, temp_arr)

    idxp = idx3.reshape(bs, 4, 32, 8)
    idx_flat = idxp.transpose(0, 2, 3, 1).reshape(N_ROWS)
    idx_wh = idxp.transpose(0, 3, 1, 2).reshape(N_ROWS)

    q = _sc_gather_kernel()(embeddings_weight, idx_wh)
    quantized = jnp.transpose(q.reshape(bs, 32, 32, EMBEDDING_DIM),
                              (0, 3, 1, 2))

    encoding_indices = idx_flat.reshape(N_ROWS, 1)
    return (quantized, loss[0, 0], perp[0, 0], encoding_indices, soft_codes)
